# baseline (speedup 1.0000x reference)
import functools

import jax
import jax.numpy as jnp
from jax import lax
from jax.experimental import pallas as pl
from jax.experimental.pallas import tpu as pltpu
from jax.experimental.pallas import tpu_sc as plsc

_N = 100000
_NNZ = 6400000
_NW = 32
_RPW = _N // _NW
_W = _NNZ // _NW
_TS = 20000
_NT = _W // _TS
_G = 10
_L = 16
_PB = 3136
_GB = 3200
_GC = _GB // 128


def _mesh():
    return plsc.VectorSubcoreMesh(core_axis_name="c", subcore_axis_name="s")


def _prefix_kernel(values):
    @functools.partial(
        pl.kernel,
        mesh=_mesh(),
        out_type=jax.ShapeDtypeStruct((_NNZ + _L,), jnp.float32),
        compiler_params=pltpu.CompilerParams(needs_layout_passes=False),
        scratch_types=[
            pltpu.VMEM((_TS,), jnp.float32),
            pltpu.VMEM((_TS,), jnp.float32),
        ],
    )
    def body(vals_hbm, s_hbm, vbuf, sbuf):
        wid = lax.axis_index("s") * 2 + lax.axis_index("c")
        base = wid * _W
        zeros = jnp.zeros((_L,), jnp.float32)
        lanes = lax.iota(jnp.int32, _L)
        gidx0 = jnp.where(lanes < _G, lanes * _L + (_L - 1), 0)

        def group_step(g, carry):
            b = g * (_G * _L)
            cs = []
            for k in range(_G):
                c = plsc.cumsum(vbuf[pl.ds(b + k * _L, _L)])
                sbuf[pl.ds(b + k * _L, _L)] = c
                cs.append(c)
            tv = plsc.cumsum(plsc.load_gather(sbuf, [gidx0 + b]))
            for k in range(_G):
                off = carry if k == 0 else carry + tv[k - 1]
                sbuf[pl.ds(b + k * _L, _L)] = cs[k] + off
            return carry + tv[_G - 1]

        def tile_step(t, carry):
            tb = base + t * _TS
            pltpu.sync_copy(vals_hbm.at[pl.ds(tb, _TS)], vbuf)
            carry = lax.fori_loop(0, _TS // (_G * _L), group_step, carry,
                                  unroll=2)
            pltpu.sync_copy(sbuf, s_hbm.at[pl.ds(tb, _TS)])
            return carry

        lax.fori_loop(0, _NT, tile_step, jnp.float32(0.0), unroll=False)
        sbuf[pl.ds(0, _L)] = zeros
        pltpu.sync_copy(sbuf.at[pl.ds(0, _L)], s_hbm.at[pl.ds(_NNZ, _L)])

    return body(values)


def _gather_diff_kernel(s_local, crow_pad):
    @functools.partial(
        pl.kernel,
        mesh=_mesh(),
        out_type=jax.ShapeDtypeStruct((_NW, _PB), jnp.float32),
        compiler_params=pltpu.CompilerParams(needs_layout_passes=False),
        scratch_types=[
            pltpu.VMEM((_PB + _L,), jnp.int32),
            pltpu.VMEM((_GB,), jnp.int32),
            pltpu.VMEM((_GB,), jnp.float32),
            pltpu.VMEM((48,), jnp.float32),
            pltpu.VMEM((48,), jnp.float32),
            pltpu.VMEM((_PB + _L,), jnp.float32),
            pltpu.VMEM((_PB,), jnp.float32),
            pltpu.SemaphoreType.DMA,
        ],
    )
    def body(s_hbm, crow_hbm, out_hbm, crowbuf, idxbuf, tgbuf, bbuf,
             cebuf, tbuf, degbuf, sem):
        wid = lax.axis_index("s") * 2 + lax.axis_index("c")
        r0 = wid * _RPW
        a0 = (r0 // 8) * 8
        off = r0 - a0
        pltpu.sync_copy(crow_hbm.at[pl.ds(a0, _PB)],
                        crowbuf.at[pl.ds(0, _PB)])

        lanes = lax.iota(jnp.int32, _L)
        zeros = jnp.zeros((_L,), jnp.float32)

        idx_lo = (lanes + 1) * _W - 1
        idx_hi = (lanes + _L + 1) * _W - 1
        pltpu.async_copy(s_hbm.at[idx_lo], cebuf.at[pl.ds(0, _L)], sem).wait()
        pltpu.async_copy(s_hbm.at[idx_hi], cebuf.at[pl.ds(_L, _L)], sem).wait()
        bbuf[pl.ds(0, _L)] = zeros
        bbuf[pl.ds(_L, _L)] = zeros
        bbuf[pl.ds(2 * _L, _L)] = zeros
        incl0 = plsc.cumsum(cebuf[pl.ds(0, _L)])
        incl1 = plsc.cumsum(cebuf[pl.ds(_L, _L)]) + incl0[_L - 1]
        plsc.store_scatter(bbuf, [lanes + 1], incl0)
        plsc.store_scatter(bbuf, [lanes + _L + 1], incl1)

        def build_idx(j, _):
            cv = crowbuf[pl.ds(off + j * _L, _L)]
            safe = cv > 0
            idxbuf[pl.ds(j * _L, _L)] = jnp.where(safe, cv - 1, _NNZ)
            return 0

        lax.fori_loop(0, _PB // _L, build_idx, 0, unroll=8)
        nnz_vec = jnp.full((_L,), _NNZ, jnp.int32)
        for k in range((_GB - _PB) // _L):
            idxbuf[pl.ds(_PB + k * _L, _L)] = nnz_vec

        def gather_step(g, _):
            b = g * (5 * 128)
            cps = [
                pltpu.async_copy(
                    s_hbm.at[idxbuf.at[pl.ds(b + k * 128, 128)]],
                    tgbuf.at[pl.ds(b + k * 128, 128)], sem)
                for k in range(5)
            ]
            for cp in cps:
                cp.wait()
            return 0

        lax.fori_loop(0, _GC // 5, gather_step, 0, unroll=False)

        def fixup(j, _):
            cv = crowbuf[pl.ds(off + j * _L, _L)]
            safe = cv > 0
            cid = jnp.where(safe, (cv - 1) // _W, 33)
            tbuf[pl.ds(j * _L, _L)] = (tgbuf[pl.ds(j * _L, _L)]
                                       + plsc.load_gather(bbuf, [cid]))
            return 0

        lax.fori_loop(0, _PB // _L, fixup, 0, unroll=8)

        def diff(j, _):
            hi = tbuf[pl.ds(j * _L + 1, _L)]
            lo = tbuf[pl.ds(j * _L, _L)]
            degbuf[pl.ds(j * _L, _L)] = hi - lo
            return 0

        lax.fori_loop(0, _PB // _L, diff, 0, unroll=8)

        pltpu.sync_copy(degbuf, out_hbm.at[wid])

    return body(s_local, crow_pad)


def kernel(crow_indices, col_indices, values):
    del col_indices
    crow_pad = jnp.pad(crow_indices, (0, 15))
    s_local = _prefix_kernel(values)
    deg_pad = _gather_diff_kernel(s_local, crow_pad)
    deg1 = deg_pad[:, :_RPW].reshape(-1).astype(jnp.float32)
    deg2 = deg1.reshape(_N, 1).astype(jnp.float32)
    return jnp.allclose(deg1, deg2.reshape(-1))


# speedup vs baseline: 1.1187x; 1.1187x over previous
import functools

import jax
import jax.numpy as jnp
from jax import lax
from jax.experimental import pallas as pl
from jax.experimental.pallas import tpu as pltpu
from jax.experimental.pallas import tpu_sc as plsc

_N = 100000
_NNZ = 6400000
_NW = 32
_RPW = _N // _NW
_W = _NNZ // _NW
_TS = 20000
_NT = _W // _TS
_L = 16
_SUB = _TS // _L
_PB = 3136
_GB = 3200
_GC = _GB // 128


def _mesh():
    return plsc.VectorSubcoreMesh(core_axis_name="c", subcore_axis_name="s")


def _prefix_kernel(values):
    @functools.partial(
        pl.kernel,
        mesh=_mesh(),
        out_type=jax.ShapeDtypeStruct((_NNZ + _L,), jnp.float32),
        compiler_params=pltpu.CompilerParams(needs_layout_passes=False),
        scratch_types=[
            pltpu.VMEM((_TS,), jnp.float32),
            pltpu.VMEM((_TS,), jnp.float32),
        ],
    )
    def body(vals_hbm, s_hbm, vbuf, sbuf):
        wid = lax.axis_index("s") * 2 + lax.axis_index("c")
        base = wid * _W
        zeros = jnp.zeros((_L,), jnp.float32)
        lanes = lax.iota(jnp.int32, _L)
        sub_idx = lanes * _SUB

        def tile_step(t, carry):
            tb = base + t * _TS
            pltpu.sync_copy(vals_hbm.at[pl.ds(tb, _TS)], vbuf)

            def p1(i, acc):
                acc = acc + plsc.load_gather(vbuf, [sub_idx + i])
                plsc.store_scatter(sbuf, [sub_idx + i], acc)
                return acc

            acc = lax.fori_loop(0, _SUB, p1, zeros, unroll=8)
            tv = plsc.cumsum(acc)
            offs = carry + tv - acc

            def p2(i, _):
                v = plsc.load_gather(sbuf, [sub_idx + i]) + offs
                plsc.store_scatter(sbuf, [sub_idx + i], v)
                return 0

            lax.fori_loop(0, _SUB, p2, 0, unroll=8)
            pltpu.sync_copy(sbuf, s_hbm.at[pl.ds(tb, _TS)])
            return carry + tv[_L - 1]

        lax.fori_loop(0, _NT, tile_step, jnp.float32(0.0), unroll=False)
        sbuf[pl.ds(0, _L)] = zeros
        pltpu.sync_copy(sbuf.at[pl.ds(0, _L)], s_hbm.at[pl.ds(_NNZ, _L)])

    return body(values)


def _gather_diff_kernel(s_local, crow_pad):
    @functools.partial(
        pl.kernel,
        mesh=_mesh(),
        out_type=jax.ShapeDtypeStruct((_NW, _PB), jnp.float32),
        compiler_params=pltpu.CompilerParams(needs_layout_passes=False),
        scratch_types=[
            pltpu.VMEM((_PB + _L,), jnp.int32),
            pltpu.VMEM((_GB,), jnp.int32),
            pltpu.VMEM((_GB,), jnp.float32),
            pltpu.VMEM((48,), jnp.float32),
            pltpu.VMEM((48,), jnp.float32),
            pltpu.VMEM((_PB + _L,), jnp.float32),
            pltpu.VMEM((_PB,), jnp.float32),
            pltpu.SemaphoreType.DMA,
        ],
    )
    def body(s_hbm, crow_hbm, out_hbm, crowbuf, idxbuf, tgbuf, bbuf,
             cebuf, tbuf, degbuf, sem):
        wid = lax.axis_index("s") * 2 + lax.axis_index("c")
        r0 = wid * _RPW
        a0 = (r0 // 8) * 8
        off = r0 - a0
        pltpu.sync_copy(crow_hbm.at[pl.ds(a0, _PB)],
                        crowbuf.at[pl.ds(0, _PB)])

        lanes = lax.iota(jnp.int32, _L)
        zeros = jnp.zeros((_L,), jnp.float32)

        idx_lo = (lanes + 1) * _W - 1
        idx_hi = (lanes + _L + 1) * _W - 1
        pltpu.async_copy(s_hbm.at[idx_lo], cebuf.at[pl.ds(0, _L)], sem).wait()
        pltpu.async_copy(s_hbm.at[idx_hi], cebuf.at[pl.ds(_L, _L)], sem).wait()
        bbuf[pl.ds(0, _L)] = zeros
        bbuf[pl.ds(_L, _L)] = zeros
        bbuf[pl.ds(2 * _L, _L)] = zeros
        incl0 = plsc.cumsum(cebuf[pl.ds(0, _L)])
        incl1 = plsc.cumsum(cebuf[pl.ds(_L, _L)]) + incl0[_L - 1]
        plsc.store_scatter(bbuf, [lanes + 1], incl0)
        plsc.store_scatter(bbuf, [lanes + _L + 1], incl1)

        def build_idx(j, _):
            cv = crowbuf[pl.ds(off + j * _L, _L)]
            safe = cv > 0
            idxbuf[pl.ds(j * _L, _L)] = jnp.where(safe, cv - 1, _NNZ)
            return 0

        lax.fori_loop(0, _PB // _L, build_idx, 0, unroll=8)
        nnz_vec = jnp.full((_L,), _NNZ, jnp.int32)
        for k in range((_GB - _PB) // _L):
            idxbuf[pl.ds(_PB + k * _L, _L)] = nnz_vec

        def gather_step(g, _):
            b = g * (5 * 128)
            cps = [
                pltpu.async_copy(
                    s_hbm.at[idxbuf.at[pl.ds(b + k * 128, 128)]],
                    tgbuf.at[pl.ds(b + k * 128, 128)], sem)
                for k in range(5)
            ]
            for cp in cps:
                cp.wait()
            return 0

        lax.fori_loop(0, _GC // 5, gather_step, 0, unroll=False)

        def fixup(j, _):
            cv = crowbuf[pl.ds(off + j * _L, _L)]
            safe = cv > 0
            cid = jnp.where(safe, (cv - 1) // _W, 33)
            tbuf[pl.ds(j * _L, _L)] = (tgbuf[pl.ds(j * _L, _L)]
                                       + plsc.load_gather(bbuf, [cid]))
            return 0

        lax.fori_loop(0, _PB // _L, fixup, 0, unroll=8)

        def diff(j, _):
            hi = tbuf[pl.ds(j * _L + 1, _L)]
            lo = tbuf[pl.ds(j * _L, _L)]
            degbuf[pl.ds(j * _L, _L)] = hi - lo
            return 0

        lax.fori_loop(0, _PB // _L, diff, 0, unroll=8)

        pltpu.sync_copy(degbuf, out_hbm.at[wid])

    return body(s_local, crow_pad)


def kernel(crow_indices, col_indices, values):
    del col_indices
    crow_pad = jnp.pad(crow_indices, (0, 15))
    s_local = _prefix_kernel(values)
    deg_pad = _gather_diff_kernel(s_local, crow_pad)
    deg1 = deg_pad[:, :_RPW].reshape(-1).astype(jnp.float32)
    deg2 = deg1.reshape(_N, 1).astype(jnp.float32)
    return jnp.allclose(deg1, deg2.reshape(-1))
